# baseline (speedup 1.0000x reference)
import jax
import jax.numpy as jnp
from jax import lax
from jax.experimental import pallas as pl
from jax.experimental.pallas import tpu as pltpu
from jax.experimental.pallas import tpu_sc as plsc

NC = 2
NS = 16
NW = NC * NS
L = 16
G = 128
WBLK = 512
CHUNK = 256


def _regroup_body(in_ref, out_ref):
    x = in_ref[...]
    out_ref[...] = (
        x.reshape(32, WBLK // 4, 4).transpose(1, 2, 0).reshape(WBLK // 4, G)
    )


def _regroup(tab_t):
    v = tab_t.shape[1]
    grid = (v + WBLK - 1) // WBLK
    return pl.pallas_call(
        _regroup_body,
        grid=(grid,),
        in_specs=[pl.BlockSpec((32, WBLK), lambda i: (0, i))],
        out_specs=pl.BlockSpec((WBLK // 4, G), lambda i: (i, 0)),
        out_shape=jax.ShapeDtypeStruct((grid * WBLK // 4, G), jnp.float32),
    )(tab_t)


def _lane_consts():
    lane = lax.iota(jnp.int32, L)
    conds = {k: (lane & k) == 0 for k in (8, 4, 2, 1)}
    perms = {k: lane ^ k for k in (8, 4, 2, 1)}
    bitrev = ((lane & 1) << 3) | ((lane & 2) << 1) | ((lane & 4) >> 1) | (
        (lane & 8) >> 3)
    return conds, perms, bitrev


def _hsum16(vecs, conds, perms, bitrev):
    for k in (8, 4, 2, 1):
        cond, perm = conds[k], perms[k]
        nxt = []
        for i in range(0, len(vecs), 2):
            x, y = vecs[i], vecs[i + 1]
            a = jnp.where(cond, x, y)
            c = jnp.where(cond, y, x)
            nxt.append(a + jnp.take_along_axis(c, perm, axis=0))
        vecs = nxt
    return jnp.take_along_axis(vecs[0], bitrev, axis=0)


def _gmf_kernel(uidx_hbm, iidx_hbm, utab_hbm, itab_hbm, w_hbm, b_hbm,
                out_hbm, uidx_v, iidx_v, uq_v, iq_v, urows_v, irows_v,
                res_v, w_v, b_v, sem):
    bpw = res_v.shape[0]
    wid = lax.axis_index("s") * NC + lax.axis_index("c")
    base = wid * bpw

    pltpu.sync_copy(w_hbm, w_v)
    pltpu.sync_copy(b_hbm, b_v)
    pltpu.sync_copy(uidx_hbm.at[pl.ds(base, bpw)], uidx_v)
    pltpu.sync_copy(iidx_hbm.at[pl.ds(base, bpw)], iidx_v)

    def shift(r, carry):
        s = pl.ds(r * L, L)
        uq_v[s] = uidx_v[s] >> 2
        iq_v[s] = iidx_v[s] >> 2
        return carry

    lax.fori_loop(0, bpw // L, shift, 0)

    w_lo = w_v[pl.ds(0, L)]
    w_hi = w_v[pl.ds(L, L)]
    bvec = b_v[...]
    conds, perms, bitrev = _lane_consts()

    def chunk(c, carry):
        coff = c * CHUNK
        cu = pltpu.async_copy(utab_hbm.at[uq_v.at[pl.ds(coff, CHUNK)]],
                              urows_v, sem)
        ci = pltpu.async_copy(itab_hbm.at[iq_v.at[pl.ds(coff, CHUNK)]],
                              irows_v, sem)
        cu.wait()
        ci.wait()

        def group(r, carry2):
            off = r * L
            uoffs = (uidx_v[pl.ds(coff + off, L)] & 3) * 32
            ioffs = (iidx_v[pl.ds(coff + off, L)] & 3) * 32
            ps = []
            for j in range(L):
                row = off + j
                uc = uoffs[j]
                ic = ioffs[j]
                u_lo = urows_v[row, pl.ds(uc, L)]
                u_hi = urows_v[row, pl.ds(uc + L, L)]
                i_lo = irows_v[row, pl.ds(ic, L)]
                i_hi = irows_v[row, pl.ds(ic + L, L)]
                ps.append(u_lo * i_lo * w_lo + u_hi * i_hi * w_hi)
            z = _hsum16(ps, conds, perms, bitrev) + bvec
            res_v[pl.ds(coff + off, L)] = 1.0 / (1.0 + jnp.exp(-z))
            return carry2

        lax.fori_loop(0, CHUNK // L, group, 0)
        return carry

    lax.fori_loop(0, bpw // CHUNK, chunk, 0)
    pltpu.sync_copy(res_v, out_hbm.at[pl.ds(base, bpw)])


def kernel(user_input, item_input, user_table, item_table, W, b):
    B = user_input.shape[0]
    V, D = user_table.shape
    assert D == 2 * L and B % (NW * CHUNK) == 0
    bpw = B // NW

    uidx = user_input.astype(jnp.int32)
    iidx = item_input.astype(jnp.int32)
    utab4 = _regroup(user_table.T)
    itab4 = _regroup(item_table.T)
    w_flat = W.reshape(D).astype(jnp.float32)
    bvec = jnp.broadcast_to(b.astype(jnp.float32).reshape(1), (L,))

    mesh = plsc.VectorSubcoreMesh(core_axis_name="c", subcore_axis_name="s")
    run = pl.kernel(
        _gmf_kernel,
        out_type=jax.ShapeDtypeStruct((B,), jnp.float32),
        mesh=mesh,
        scratch_types=[
            pltpu.VMEM((bpw,), jnp.int32),
            pltpu.VMEM((bpw,), jnp.int32),
            pltpu.VMEM((bpw,), jnp.int32),
            pltpu.VMEM((bpw,), jnp.int32),
            pltpu.VMEM((CHUNK, G), jnp.float32),
            pltpu.VMEM((CHUNK, G), jnp.float32),
            pltpu.VMEM((bpw,), jnp.float32),
            pltpu.VMEM((D,), jnp.float32),
            pltpu.VMEM((L,), jnp.float32),
            pltpu.SemaphoreType.DMA,
        ],
    )
    out = run(uidx, iidx, utab4, itab4, w_flat, bvec)
    return out.reshape(B, 1)


# speedup vs baseline: 9.2082x; 9.2082x over previous
import jax
import jax.numpy as jnp
from jax import lax
from jax.experimental import pallas as pl
from jax.experimental.pallas import tpu as pltpu
from jax.experimental.pallas import tpu_sc as plsc

NC = 2
NS = 16
NW = NC * NS
L = 16
G = 128
WBLK = 4096
QB = WBLK // 4
CHUNK = 256


def _regroup_body(in_ref, out_ref):
    y = in_ref[...].T
    out_ref[...] = jnp.concatenate(
        [y[p * QB:(p + 1) * QB] for p in range(4)], axis=1)


def _regroup(tab_t):
    v = tab_t.shape[1]
    grid = (v + WBLK - 1) // WBLK
    return pl.pallas_call(
        _regroup_body,
        grid=(grid,),
        in_specs=[pl.BlockSpec((32, WBLK), lambda i: (0, i))],
        out_specs=pl.BlockSpec((QB, G), lambda i: (i, 0)),
        out_shape=jax.ShapeDtypeStruct((grid * QB, G), jnp.float32),
        compiler_params=pltpu.CompilerParams(
            dimension_semantics=("arbitrary",)),
    )(tab_t)


def _lane_consts():
    lane = lax.iota(jnp.int32, L)
    conds = {k: (lane & k) == 0 for k in (8, 4, 2, 1)}
    perms = {k: lane ^ k for k in (8, 4, 2, 1)}
    bitrev = ((lane & 1) << 3) | ((lane & 2) << 1) | ((lane & 4) >> 1) | (
        (lane & 8) >> 3)
    return conds, perms, bitrev


def _hsum16(vecs, conds, perms, bitrev):
    for k in (8, 4, 2, 1):
        cond, perm = conds[k], perms[k]
        nxt = []
        for i in range(0, len(vecs), 2):
            x, y = vecs[i], vecs[i + 1]
            a = jnp.where(cond, x, y)
            c = jnp.where(cond, y, x)
            nxt.append(a + jnp.take_along_axis(c, perm, axis=0))
        vecs = nxt
    return jnp.take_along_axis(vecs[0], bitrev, axis=0)


def _gmf_kernel(uidx_hbm, iidx_hbm, utab_hbm, itab_hbm, w_hbm, b_hbm,
                out_hbm, uidx_v, iidx_v, uq_v, iq_v, urows_v, irows_v,
                res_v, w_v, b_v, sem):
    bpw = res_v.shape[0]
    wid = lax.axis_index("s") * NC + lax.axis_index("c")
    base = wid * bpw

    pltpu.sync_copy(w_hbm, w_v)
    pltpu.sync_copy(b_hbm, b_v)
    pltpu.sync_copy(uidx_hbm.at[pl.ds(base, bpw)], uidx_v)
    pltpu.sync_copy(iidx_hbm.at[pl.ds(base, bpw)], iidx_v)

    def shift(r, carry):
        s = pl.ds(r * L, L)
        u = uidx_v[s]
        i = iidx_v[s]
        uq_v[s] = ((u >> 12) << 10) | (u & 1023)
        iq_v[s] = ((i >> 12) << 10) | (i & 1023)
        return carry

    lax.fori_loop(0, bpw // L, shift, 0)

    w_lo = w_v[pl.ds(0, L)]
    w_hi = w_v[pl.ds(L, L)]
    bvec = b_v[...]
    conds, perms, bitrev = _lane_consts()

    def chunk(c, carry):
        coff = c * CHUNK
        cu = pltpu.async_copy(utab_hbm.at[uq_v.at[pl.ds(coff, CHUNK)]],
                              urows_v, sem)
        ci = pltpu.async_copy(itab_hbm.at[iq_v.at[pl.ds(coff, CHUNK)]],
                              irows_v, sem)
        cu.wait()
        ci.wait()

        def group(r, carry2):
            off = r * L
            uoffs = ((uidx_v[pl.ds(coff + off, L)] >> 10) & 3) << 5
            ioffs = ((iidx_v[pl.ds(coff + off, L)] >> 10) & 3) << 5
            ps = []
            for j in range(L):
                row = off + j
                uc = uoffs[j]
                ic = ioffs[j]
                u_lo = urows_v[row, pl.ds(uc, L)]
                u_hi = urows_v[row, pl.ds(uc + L, L)]
                i_lo = irows_v[row, pl.ds(ic, L)]
                i_hi = irows_v[row, pl.ds(ic + L, L)]
                ps.append(u_lo * i_lo * w_lo + u_hi * i_hi * w_hi)
            z = _hsum16(ps, conds, perms, bitrev) + bvec
            res_v[pl.ds(coff + off, L)] = 1.0 / (1.0 + jnp.exp(-z))
            return carry2

        lax.fori_loop(0, CHUNK // L, group, 0)
        return carry

    lax.fori_loop(0, bpw // CHUNK, chunk, 0)
    pltpu.sync_copy(res_v, out_hbm.at[pl.ds(base, bpw)])


def kernel(user_input, item_input, user_table, item_table, W, b):
    B = user_input.shape[0]
    V, D = user_table.shape
    assert D == 2 * L and B % (NW * CHUNK) == 0
    bpw = B // NW

    uidx = user_input.astype(jnp.int32)
    iidx = item_input.astype(jnp.int32)
    utab4 = _regroup(user_table.T)
    itab4 = _regroup(item_table.T)
    w_flat = W.reshape(D).astype(jnp.float32)
    bvec = jnp.broadcast_to(b.astype(jnp.float32).reshape(1), (L,))

    mesh = plsc.VectorSubcoreMesh(core_axis_name="c", subcore_axis_name="s")
    run = pl.kernel(
        _gmf_kernel,
        out_type=jax.ShapeDtypeStruct((B,), jnp.float32),
        mesh=mesh,
        scratch_types=[
            pltpu.VMEM((bpw,), jnp.int32),
            pltpu.VMEM((bpw,), jnp.int32),
            pltpu.VMEM((bpw,), jnp.int32),
            pltpu.VMEM((bpw,), jnp.int32),
            pltpu.VMEM((CHUNK, G), jnp.float32),
            pltpu.VMEM((CHUNK, G), jnp.float32),
            pltpu.VMEM((bpw,), jnp.float32),
            pltpu.VMEM((D,), jnp.float32),
            pltpu.VMEM((L,), jnp.float32),
            pltpu.SemaphoreType.DMA,
        ],
    )
    out = run(uidx, iidx, utab4, itab4, w_flat, bvec)
    return out.reshape(B, 1)


# speedup vs baseline: 10.8980x; 1.1835x over previous
import jax
import jax.numpy as jnp
from jax import lax
from jax.experimental import pallas as pl
from jax.experimental.pallas import tpu as pltpu
from jax.experimental.pallas import tpu_sc as plsc

NC = 2
NS = 16
NW = NC * NS
L = 16
G = 128
SHW = 14
WBLK = 1 << SHW
QB = WBLK // 4
SHQ = SHW - 2
CHUNK = 256


def _regroup_body(in_ref, out_ref):
    y = in_ref[...].T
    out_ref[...] = jnp.concatenate(
        [y[p * QB:(p + 1) * QB] for p in range(4)], axis=1)


def _regroup(tab_t):
    v = tab_t.shape[1]
    grid = (v + WBLK - 1) // WBLK
    return pl.pallas_call(
        _regroup_body,
        grid=(grid,),
        in_specs=[pl.BlockSpec((32, WBLK), lambda i: (0, i))],
        out_specs=pl.BlockSpec((QB, G), lambda i: (i, 0)),
        out_shape=jax.ShapeDtypeStruct((grid * QB, G), jnp.float32),
        compiler_params=pltpu.CompilerParams(
            dimension_semantics=("arbitrary",)),
    )(tab_t)


def _lane_consts():
    lane = lax.iota(jnp.int32, L)
    conds = {k: (lane & k) == 0 for k in (8, 4, 2, 1)}
    perms = {k: lane ^ k for k in (8, 4, 2, 1)}
    bitrev = ((lane & 1) << 3) | ((lane & 2) << 1) | ((lane & 4) >> 1) | (
        (lane & 8) >> 3)
    return conds, perms, bitrev


def _hsum16(vecs, conds, perms, bitrev):
    for k in (8, 4, 2, 1):
        cond, perm = conds[k], perms[k]
        nxt = []
        for i in range(0, len(vecs), 2):
            x, y = vecs[i], vecs[i + 1]
            a = jnp.where(cond, x, y)
            c = jnp.where(cond, y, x)
            nxt.append(a + jnp.take_along_axis(c, perm, axis=0))
        vecs = nxt
    return jnp.take_along_axis(vecs[0], bitrev, axis=0)


def _gmf_kernel(uidx_hbm, iidx_hbm, utab_hbm, itab_hbm, w_hbm, b_hbm,
                out_hbm, uidx_v, iidx_v, uq_v, iq_v, urows_v, irows_v,
                res_v, w_v, b_v, sem):
    bpw = res_v.shape[0]
    wid = lax.axis_index("s") * NC + lax.axis_index("c")
    base = wid * bpw

    pltpu.sync_copy(w_hbm, w_v)
    pltpu.sync_copy(b_hbm, b_v)
    pltpu.sync_copy(uidx_hbm.at[pl.ds(base, bpw)], uidx_v)
    pltpu.sync_copy(iidx_hbm.at[pl.ds(base, bpw)], iidx_v)

    def shift(r, carry):
        s = pl.ds(r * L, L)
        u = uidx_v[s]
        i = iidx_v[s]
        uq_v[s] = ((u >> SHW) << SHQ) | (u & (QB - 1))
        iq_v[s] = ((i >> SHW) << SHQ) | (i & (QB - 1))
        return carry

    lax.fori_loop(0, bpw // L, shift, 0)

    w_lo = w_v[pl.ds(0, L)]
    w_hi = w_v[pl.ds(L, L)]
    bvec = b_v[...]
    conds, perms, bitrev = _lane_consts()

    def chunk(c, carry):
        coff = c * CHUNK
        cu = pltpu.async_copy(utab_hbm.at[uq_v.at[pl.ds(coff, CHUNK)]],
                              urows_v, sem)
        ci = pltpu.async_copy(itab_hbm.at[iq_v.at[pl.ds(coff, CHUNK)]],
                              irows_v, sem)
        cu.wait()
        ci.wait()

        def group(r, carry2):
            off = r * L
            uoffs = ((uidx_v[pl.ds(coff + off, L)] >> SHQ) & 3) << 5
            ioffs = ((iidx_v[pl.ds(coff + off, L)] >> SHQ) & 3) << 5
            ps = []
            for j in range(L):
                row = off + j
                uc = uoffs[j]
                ic = ioffs[j]
                u_lo = urows_v[row, pl.ds(uc, L)]
                u_hi = urows_v[row, pl.ds(uc + L, L)]
                i_lo = irows_v[row, pl.ds(ic, L)]
                i_hi = irows_v[row, pl.ds(ic + L, L)]
                ps.append(u_lo * i_lo * w_lo + u_hi * i_hi * w_hi)
            z = _hsum16(ps, conds, perms, bitrev) + bvec
            res_v[pl.ds(coff + off, L)] = 1.0 / (1.0 + jnp.exp(-z))
            return carry2

        lax.fori_loop(0, CHUNK // L, group, 0)
        return carry

    lax.fori_loop(0, bpw // CHUNK, chunk, 0)
    pltpu.sync_copy(res_v, out_hbm.at[pl.ds(base, bpw)])


def kernel(user_input, item_input, user_table, item_table, W, b):
    B = user_input.shape[0]
    V, D = user_table.shape
    assert D == 2 * L and B % (NW * CHUNK) == 0
    bpw = B // NW

    uidx = user_input.astype(jnp.int32)
    iidx = item_input.astype(jnp.int32)
    utab4 = _regroup(user_table.T)
    itab4 = _regroup(item_table.T)
    w_flat = W.reshape(D).astype(jnp.float32)
    bvec = jnp.broadcast_to(b.astype(jnp.float32).reshape(1), (L,))

    mesh = plsc.VectorSubcoreMesh(core_axis_name="c", subcore_axis_name="s")
    run = pl.kernel(
        _gmf_kernel,
        out_type=jax.ShapeDtypeStruct((B,), jnp.float32),
        mesh=mesh,
        scratch_types=[
            pltpu.VMEM((bpw,), jnp.int32),
            pltpu.VMEM((bpw,), jnp.int32),
            pltpu.VMEM((bpw,), jnp.int32),
            pltpu.VMEM((bpw,), jnp.int32),
            pltpu.VMEM((CHUNK, G), jnp.float32),
            pltpu.VMEM((CHUNK, G), jnp.float32),
            pltpu.VMEM((bpw,), jnp.float32),
            pltpu.VMEM((D,), jnp.float32),
            pltpu.VMEM((L,), jnp.float32),
            pltpu.SemaphoreType.DMA,
        ],
    )
    out = run(uidx, iidx, utab4, itab4, w_flat, bvec)
    return out.reshape(B, 1)


# speedup vs baseline: 24.7832x; 2.2741x over previous
import jax
import jax.numpy as jnp
from jax import lax
from jax.experimental import pallas as pl
from jax.experimental.pallas import tpu as pltpu
from jax.experimental.pallas import tpu_sc as plsc

NC = 2
NS = 16
NW = NC * NS
L = 16
G = 128
SHW = 14
WBLK = 1 << SHW
QB = WBLK // 4
SHQ = SHW - 2
CHUNK = 256


def _regroup_body(in_ref, out_ref):
    x = in_ref[...]
    xt = jnp.concatenate(
        [x[:, p * QB:(p + 1) * QB] for p in range(4)], axis=0)
    out_ref[...] = xt.T


def _regroup(tab_t):
    v = tab_t.shape[1]
    grid = (v + WBLK - 1) // WBLK
    return pl.pallas_call(
        _regroup_body,
        grid=(grid,),
        in_specs=[pl.BlockSpec((32, WBLK), lambda i: (0, i))],
        out_specs=pl.BlockSpec((QB, G), lambda i: (i, 0)),
        out_shape=jax.ShapeDtypeStruct((grid * QB, G), jnp.float32),
        compiler_params=pltpu.CompilerParams(
            dimension_semantics=("arbitrary",)),
    )(tab_t)


def _lane_consts():
    lane = lax.iota(jnp.int32, L)
    conds = {k: (lane & k) == 0 for k in (8, 4, 2, 1)}
    perms = {k: lane ^ k for k in (8, 4, 2, 1)}
    bitrev = ((lane & 1) << 3) | ((lane & 2) << 1) | ((lane & 4) >> 1) | (
        (lane & 8) >> 3)
    return conds, perms, bitrev


def _hsum16(vecs, conds, perms, bitrev):
    for k in (8, 4, 2, 1):
        cond, perm = conds[k], perms[k]
        nxt = []
        for i in range(0, len(vecs), 2):
            x, y = vecs[i], vecs[i + 1]
            a = jnp.where(cond, x, y)
            c = jnp.where(cond, y, x)
            nxt.append(a + jnp.take_along_axis(c, perm, axis=0))
        vecs = nxt
    return jnp.take_along_axis(vecs[0], bitrev, axis=0)


def _gmf_kernel(uidx_hbm, iidx_hbm, utab_hbm, itab_hbm, w_hbm, b_hbm,
                out_hbm, uidx_v, iidx_v, uq_v, iq_v, urows_v, irows_v,
                res_v, w_v, b_v, sem):
    bpw = res_v.shape[0]
    wid = lax.axis_index("s") * NC + lax.axis_index("c")
    base = wid * bpw

    pltpu.sync_copy(w_hbm, w_v)
    pltpu.sync_copy(b_hbm, b_v)
    pltpu.sync_copy(uidx_hbm.at[pl.ds(base, bpw)], uidx_v)
    pltpu.sync_copy(iidx_hbm.at[pl.ds(base, bpw)], iidx_v)

    def shift(r, carry):
        s = pl.ds(r * L, L)
        u = uidx_v[s]
        i = iidx_v[s]
        uq_v[s] = ((u >> SHW) << SHQ) | (u & (QB - 1))
        iq_v[s] = ((i >> SHW) << SHQ) | (i & (QB - 1))
        return carry

    lax.fori_loop(0, bpw // L, shift, 0)

    w_lo = w_v[pl.ds(0, L)]
    w_hi = w_v[pl.ds(L, L)]
    bvec = b_v[...]
    conds, perms, bitrev = _lane_consts()

    def chunk(c, carry):
        coff = c * CHUNK
        cu = pltpu.async_copy(utab_hbm.at[uq_v.at[pl.ds(coff, CHUNK)]],
                              urows_v, sem)
        ci = pltpu.async_copy(itab_hbm.at[iq_v.at[pl.ds(coff, CHUNK)]],
                              irows_v, sem)
        cu.wait()
        ci.wait()

        def group(r, carry2):
            off = r * L
            uoffs = ((uidx_v[pl.ds(coff + off, L)] >> SHQ) & 3) << 5
            ioffs = ((iidx_v[pl.ds(coff + off, L)] >> SHQ) & 3) << 5
            ps = []
            for j in range(L):
                row = off + j
                uc = uoffs[j]
                ic = ioffs[j]
                u_lo = urows_v[row, pl.ds(uc, L)]
                u_hi = urows_v[row, pl.ds(uc + L, L)]
                i_lo = irows_v[row, pl.ds(ic, L)]
                i_hi = irows_v[row, pl.ds(ic + L, L)]
                ps.append(u_lo * i_lo * w_lo + u_hi * i_hi * w_hi)
            z = _hsum16(ps, conds, perms, bitrev) + bvec
            res_v[pl.ds(coff + off, L)] = 1.0 / (1.0 + jnp.exp(-z))
            return carry2

        lax.fori_loop(0, CHUNK // L, group, 0)
        return carry

    lax.fori_loop(0, bpw // CHUNK, chunk, 0)
    pltpu.sync_copy(res_v, out_hbm.at[pl.ds(base, bpw)])


def kernel(user_input, item_input, user_table, item_table, W, b):
    B = user_input.shape[0]
    V, D = user_table.shape
    assert D == 2 * L and B % (NW * CHUNK) == 0
    bpw = B // NW

    uidx = user_input.astype(jnp.int32)
    iidx = item_input.astype(jnp.int32)
    utab4 = _regroup(user_table.T)
    itab4 = _regroup(item_table.T)
    w_flat = W.reshape(D).astype(jnp.float32)
    bvec = jnp.broadcast_to(b.astype(jnp.float32).reshape(1), (L,))

    mesh = plsc.VectorSubcoreMesh(core_axis_name="c", subcore_axis_name="s")
    run = pl.kernel(
        _gmf_kernel,
        out_type=jax.ShapeDtypeStruct((B,), jnp.float32),
        mesh=mesh,
        scratch_types=[
            pltpu.VMEM((bpw,), jnp.int32),
            pltpu.VMEM((bpw,), jnp.int32),
            pltpu.VMEM((bpw,), jnp.int32),
            pltpu.VMEM((bpw,), jnp.int32),
            pltpu.VMEM((CHUNK, G), jnp.float32),
            pltpu.VMEM((CHUNK, G), jnp.float32),
            pltpu.VMEM((bpw,), jnp.float32),
            pltpu.VMEM((D,), jnp.float32),
            pltpu.VMEM((L,), jnp.float32),
            pltpu.SemaphoreType.DMA,
        ],
    )
    out = run(uidx, iidx, utab4, itab4, w_flat, bvec)
    return out.reshape(B, 1)


# speedup vs baseline: 28.2310x; 1.1391x over previous
import jax
import jax.numpy as jnp
from jax import lax
from jax.experimental import pallas as pl
from jax.experimental.pallas import tpu as pltpu
from jax.experimental.pallas import tpu_sc as plsc

NC = 2
NS = 16
NW = NC * NS
L = 16
G = 128
SHW = 15
WBLK = 1 << SHW
QB = WBLK // 4
SHQ = SHW - 2
CHUNK = 256


def _regroup_body(in_ref, out_ref):
    x = in_ref[...]
    xt = jnp.concatenate(
        [x[:, p * QB:(p + 1) * QB] for p in range(4)], axis=0)
    out_ref[...] = xt.T


def _regroup(tab_t):
    v = tab_t.shape[1]
    grid = (v + WBLK - 1) // WBLK
    return pl.pallas_call(
        _regroup_body,
        grid=(grid,),
        in_specs=[pl.BlockSpec((32, WBLK), lambda i: (0, i))],
        out_specs=pl.BlockSpec((QB, G), lambda i: (i, 0)),
        out_shape=jax.ShapeDtypeStruct((grid * QB, G), jnp.float32),
        compiler_params=pltpu.CompilerParams(
            dimension_semantics=("arbitrary",)),
    )(tab_t)


def _lane_consts():
    lane = lax.iota(jnp.int32, L)
    conds = {k: (lane & k) == 0 for k in (8, 4, 2, 1)}
    perms = {k: lane ^ k for k in (8, 4, 2, 1)}
    bitrev = ((lane & 1) << 3) | ((lane & 2) << 1) | ((lane & 4) >> 1) | (
        (lane & 8) >> 3)
    return conds, perms, bitrev


def _hsum16(vecs, conds, perms, bitrev):
    for k in (8, 4, 2, 1):
        cond, perm = conds[k], perms[k]
        nxt = []
        for i in range(0, len(vecs), 2):
            x, y = vecs[i], vecs[i + 1]
            a = jnp.where(cond, x, y)
            c = jnp.where(cond, y, x)
            nxt.append(a + jnp.take_along_axis(c, perm, axis=0))
        vecs = nxt
    return jnp.take_along_axis(vecs[0], bitrev, axis=0)


def _gmf_kernel(uidx_hbm, iidx_hbm, utab_hbm, itab_hbm, w_hbm, b_hbm,
                out_hbm, uidx_v, iidx_v, uq_v, iq_v, urows_v, irows_v,
                res_v, w_v, b_v, sem):
    bpw = res_v.shape[0]
    wid = lax.axis_index("s") * NC + lax.axis_index("c")
    base = wid * bpw

    pltpu.sync_copy(w_hbm, w_v)
    pltpu.sync_copy(b_hbm, b_v)
    pltpu.sync_copy(uidx_hbm.at[pl.ds(base, bpw)], uidx_v)
    pltpu.sync_copy(iidx_hbm.at[pl.ds(base, bpw)], iidx_v)

    def shift(r, carry):
        s = pl.ds(r * L, L)
        u = uidx_v[s]
        i = iidx_v[s]
        uq_v[s] = ((u >> SHW) << SHQ) | (u & (QB - 1))
        iq_v[s] = ((i >> SHW) << SHQ) | (i & (QB - 1))
        return carry

    lax.fori_loop(0, bpw // L, shift, 0)

    w_lo = w_v[pl.ds(0, L)]
    w_hi = w_v[pl.ds(L, L)]
    bvec = b_v[...]
    conds, perms, bitrev = _lane_consts()

    def chunk(c, carry):
        coff = c * CHUNK
        cu = pltpu.async_copy(utab_hbm.at[uq_v.at[pl.ds(coff, CHUNK)]],
                              urows_v, sem)
        ci = pltpu.async_copy(itab_hbm.at[iq_v.at[pl.ds(coff, CHUNK)]],
                              irows_v, sem)
        cu.wait()
        ci.wait()

        def group(r, carry2):
            off = r * L
            uoffs = ((uidx_v[pl.ds(coff + off, L)] >> SHQ) & 3) << 5
            ioffs = ((iidx_v[pl.ds(coff + off, L)] >> SHQ) & 3) << 5
            ps = []
            for j in range(L):
                row = off + j
                uc = uoffs[j]
                ic = ioffs[j]
                u_lo = urows_v[row, pl.ds(uc, L)]
                u_hi = urows_v[row, pl.ds(uc + L, L)]
                i_lo = irows_v[row, pl.ds(ic, L)]
                i_hi = irows_v[row, pl.ds(ic + L, L)]
                ps.append(u_lo * i_lo * w_lo + u_hi * i_hi * w_hi)
            z = _hsum16(ps, conds, perms, bitrev) + bvec
            res_v[pl.ds(coff + off, L)] = 1.0 / (1.0 + jnp.exp(-z))
            return carry2

        lax.fori_loop(0, CHUNK // L, group, 0)
        return carry

    lax.fori_loop(0, bpw // CHUNK, chunk, 0)
    pltpu.sync_copy(res_v, out_hbm.at[pl.ds(base, bpw)])


def kernel(user_input, item_input, user_table, item_table, W, b):
    B = user_input.shape[0]
    V, D = user_table.shape
    assert D == 2 * L and B % (NW * CHUNK) == 0
    bpw = B // NW

    uidx = user_input.astype(jnp.int32)
    iidx = item_input.astype(jnp.int32)
    utab4 = _regroup(user_table.T)
    itab4 = _regroup(item_table.T)
    w_flat = W.reshape(D).astype(jnp.float32)
    bvec = jnp.broadcast_to(b.astype(jnp.float32).reshape(1), (L,))

    mesh = plsc.VectorSubcoreMesh(core_axis_name="c", subcore_axis_name="s")
    run = pl.kernel(
        _gmf_kernel,
        out_type=jax.ShapeDtypeStruct((B,), jnp.float32),
        mesh=mesh,
        scratch_types=[
            pltpu.VMEM((bpw,), jnp.int32),
            pltpu.VMEM((bpw,), jnp.int32),
            pltpu.VMEM((bpw,), jnp.int32),
            pltpu.VMEM((bpw,), jnp.int32),
            pltpu.VMEM((CHUNK, G), jnp.float32),
            pltpu.VMEM((CHUNK, G), jnp.float32),
            pltpu.VMEM((bpw,), jnp.float32),
            pltpu.VMEM((D,), jnp.float32),
            pltpu.VMEM((L,), jnp.float32),
            pltpu.SemaphoreType.DMA,
        ],
    )
    out = run(uidx, iidx, utab4, itab4, w_flat, bvec)
    return out.reshape(B, 1)


# speedup vs baseline: 28.6065x; 1.0133x over previous
import jax
import jax.numpy as jnp
from jax import lax
from jax.experimental import pallas as pl
from jax.experimental.pallas import tpu as pltpu
from jax.experimental.pallas import tpu_sc as plsc

NC = 2
NS = 16
NW = NC * NS
L = 16
G = 128
SHW = 16
WBLK = 1 << SHW
QB = WBLK // 4
SHQ = SHW - 2
CHUNK = 256


def _regroup_body(in_ref, out_ref):
    x = in_ref[...]
    xt = jnp.concatenate(
        [x[:, p * QB:(p + 1) * QB] for p in range(4)], axis=0)
    out_ref[...] = xt.T


def _regroup(tab_t):
    v = tab_t.shape[1]
    grid = (v + WBLK - 1) // WBLK
    return pl.pallas_call(
        _regroup_body,
        grid=(grid,),
        in_specs=[pl.BlockSpec((32, WBLK), lambda i: (0, i))],
        out_specs=pl.BlockSpec((QB, G), lambda i: (i, 0)),
        out_shape=jax.ShapeDtypeStruct((grid * QB, G), jnp.float32),
        compiler_params=pltpu.CompilerParams(
            dimension_semantics=("arbitrary",)),
    )(tab_t)


def _lane_consts():
    lane = lax.iota(jnp.int32, L)
    conds = {k: (lane & k) == 0 for k in (8, 4, 2, 1)}
    perms = {k: lane ^ k for k in (8, 4, 2, 1)}
    bitrev = ((lane & 1) << 3) | ((lane & 2) << 1) | ((lane & 4) >> 1) | (
        (lane & 8) >> 3)
    return conds, perms, bitrev


def _hsum16(vecs, conds, perms, bitrev):
    for k in (8, 4, 2, 1):
        cond, perm = conds[k], perms[k]
        nxt = []
        for i in range(0, len(vecs), 2):
            x, y = vecs[i], vecs[i + 1]
            a = jnp.where(cond, x, y)
            c = jnp.where(cond, y, x)
            nxt.append(a + jnp.take_along_axis(c, perm, axis=0))
        vecs = nxt
    return jnp.take_along_axis(vecs[0], bitrev, axis=0)


def _gmf_kernel(uidx_hbm, iidx_hbm, utab_hbm, itab_hbm, w_hbm, b_hbm,
                out_hbm, uidx_v, iidx_v, uq_v, iq_v, urows_v, irows_v,
                res_v, w_v, b_v, sem):
    bpw = res_v.shape[0]
    wid = lax.axis_index("s") * NC + lax.axis_index("c")
    base = wid * bpw

    pltpu.sync_copy(w_hbm, w_v)
    pltpu.sync_copy(b_hbm, b_v)
    pltpu.sync_copy(uidx_hbm.at[pl.ds(base, bpw)], uidx_v)
    pltpu.sync_copy(iidx_hbm.at[pl.ds(base, bpw)], iidx_v)

    def shift(r, carry):
        s = pl.ds(r * L, L)
        u = uidx_v[s]
        i = iidx_v[s]
        uq_v[s] = ((u >> SHW) << SHQ) | (u & (QB - 1))
        iq_v[s] = ((i >> SHW) << SHQ) | (i & (QB - 1))
        return carry

    lax.fori_loop(0, bpw // L, shift, 0)

    w_lo = w_v[pl.ds(0, L)]
    w_hi = w_v[pl.ds(L, L)]
    bvec = b_v[...]
    conds, perms, bitrev = _lane_consts()

    def chunk(c, carry):
        coff = c * CHUNK
        cu = pltpu.async_copy(utab_hbm.at[uq_v.at[pl.ds(coff, CHUNK)]],
                              urows_v, sem)
        ci = pltpu.async_copy(itab_hbm.at[iq_v.at[pl.ds(coff, CHUNK)]],
                              irows_v, sem)
        cu.wait()
        ci.wait()

        def group(r, carry2):
            off = r * L
            uoffs = ((uidx_v[pl.ds(coff + off, L)] >> SHQ) & 3) << 5
            ioffs = ((iidx_v[pl.ds(coff + off, L)] >> SHQ) & 3) << 5
            ps = []
            for j in range(L):
                row = off + j
                uc = uoffs[j]
                ic = ioffs[j]
                u_lo = urows_v[row, pl.ds(uc, L)]
                u_hi = urows_v[row, pl.ds(uc + L, L)]
                i_lo = irows_v[row, pl.ds(ic, L)]
                i_hi = irows_v[row, pl.ds(ic + L, L)]
                ps.append(u_lo * i_lo * w_lo + u_hi * i_hi * w_hi)
            z = _hsum16(ps, conds, perms, bitrev) + bvec
            res_v[pl.ds(coff + off, L)] = 1.0 / (1.0 + jnp.exp(-z))
            return carry2

        lax.fori_loop(0, CHUNK // L, group, 0)
        return carry

    lax.fori_loop(0, bpw // CHUNK, chunk, 0)
    pltpu.sync_copy(res_v, out_hbm.at[pl.ds(base, bpw)])


def kernel(user_input, item_input, user_table, item_table, W, b):
    B = user_input.shape[0]
    V, D = user_table.shape
    assert D == 2 * L and B % (NW * CHUNK) == 0
    bpw = B // NW

    uidx = user_input.astype(jnp.int32)
    iidx = item_input.astype(jnp.int32)
    utab4 = _regroup(user_table.T)
    itab4 = _regroup(item_table.T)
    w_flat = W.reshape(D).astype(jnp.float32)
    bvec = jnp.broadcast_to(b.astype(jnp.float32).reshape(1), (L,))

    mesh = plsc.VectorSubcoreMesh(core_axis_name="c", subcore_axis_name="s")
    run = pl.kernel(
        _gmf_kernel,
        out_type=jax.ShapeDtypeStruct((B,), jnp.float32),
        mesh=mesh,
        scratch_types=[
            pltpu.VMEM((bpw,), jnp.int32),
            pltpu.VMEM((bpw,), jnp.int32),
            pltpu.VMEM((bpw,), jnp.int32),
            pltpu.VMEM((bpw,), jnp.int32),
            pltpu.VMEM((CHUNK, G), jnp.float32),
            pltpu.VMEM((CHUNK, G), jnp.float32),
            pltpu.VMEM((bpw,), jnp.float32),
            pltpu.VMEM((D,), jnp.float32),
            pltpu.VMEM((L,), jnp.float32),
            pltpu.SemaphoreType.DMA,
        ],
    )
    out = run(uidx, iidx, utab4, itab4, w_flat, bvec)
    return out.reshape(B, 1)
